# baseline (speedup 1.0000x reference)
import functools

import jax
import jax.numpy as jnp
from jax import lax
from jax.experimental import pallas as pl
from jax.experimental.pallas import tpu as pltpu
from jax.experimental.pallas import tpu_sc as plsc

N = 1_000_000
NC = 2
NS = 16
NW = NC * NS
L = 16

CHUNK = (N // NW) // 8 * 8
TAIL = N - CHUNK * NW
TBL = 32


def _scale_shift_body(x_hbm, z_hbm, scale_hbm, shift_hbm, out_hbm,
                      x_v, z_v, o_v, scale_v, shift_v, tx_v, tz_v, to_v):
    wid = lax.axis_index("s") * NC + lax.axis_index("c")
    base = wid * CHUNK

    pltpu.sync_copy(scale_hbm, scale_v)
    pltpu.sync_copy(shift_hbm, shift_v)

    pltpu.sync_copy(x_hbm.at[pl.ds(base, CHUNK)], x_v)
    pltpu.sync_copy(z_hbm.at[pl.ds(base, CHUNK)], z_v)

    def body(i, _):
        off = i * L
        zv = z_v[pl.ds(off, L)]
        xv = x_v[pl.ds(off, L)]
        sv = plsc.load_gather(scale_v, [zv])
        tv = plsc.load_gather(shift_v, [zv])
        o_v[pl.ds(off, L)] = xv * sv + tv
        return _

    lax.fori_loop(0, CHUNK // L, body, None, unroll=8)

    pltpu.sync_copy(o_v, out_hbm.at[pl.ds(base, CHUNK)])

    @pl.when(wid == NW - 1)
    def _():
        tbase = CHUNK * NW
        pltpu.sync_copy(x_hbm.at[pl.ds(tbase, TAIL)], tx_v)
        pltpu.sync_copy(z_hbm.at[pl.ds(tbase, TAIL)], tz_v)

        def tbody(i, _):
            off = i * L
            zv = tz_v[pl.ds(off, L)]
            xv = tx_v[pl.ds(off, L)]
            sv = plsc.load_gather(scale_v, [zv])
            tv = plsc.load_gather(shift_v, [zv])
            to_v[pl.ds(off, L)] = xv * sv + tv
            return _

        lax.fori_loop(0, TAIL // L, tbody, None, unroll=TAIL // L)
        pltpu.sync_copy(to_v, out_hbm.at[pl.ds(tbase, TAIL)])


@jax.jit
def kernel(inputs, z, scale_table, shift_table):
    x = inputs.reshape(N)
    zi = z.astype(jnp.int32)
    zmax = scale_table.shape[0]
    scale = jnp.zeros((TBL,), jnp.float32).at[:zmax].set(scale_table.reshape(-1))
    shift = jnp.zeros((TBL,), jnp.float32).at[:zmax].set(shift_table.reshape(-1))

    mesh = plsc.VectorSubcoreMesh(core_axis_name="c", subcore_axis_name="s",
                                  num_cores=NC, num_subcores=NS)
    out = pl.kernel(
        _scale_shift_body,
        out_type=jax.ShapeDtypeStruct((N,), jnp.float32),
        mesh=mesh,
        compiler_params=pltpu.CompilerParams(needs_layout_passes=False),
        scratch_types=[
            pltpu.VMEM((CHUNK,), jnp.float32),
            pltpu.VMEM((CHUNK,), jnp.int32),
            pltpu.VMEM((CHUNK,), jnp.float32),
            pltpu.VMEM((TBL,), jnp.float32),
            pltpu.VMEM((TBL,), jnp.float32),
            pltpu.VMEM((TAIL,), jnp.float32),
            pltpu.VMEM((TAIL,), jnp.int32),
            pltpu.VMEM((TAIL,), jnp.float32),
        ],
    )(x, zi, scale, shift)
    return out.reshape(N, 1)


# speedup vs baseline: 1.5495x; 1.5495x over previous
import jax
import jax.numpy as jnp
from jax import lax
from jax.experimental import pallas as pl
from jax.experimental.pallas import tpu as pltpu
from jax.experimental.pallas import tpu_sc as plsc

N = 1_000_000
NC = 2
NS = 16
NW = NC * NS
L = 16

CHUNK = (N // NW) // 8 * 8
TAIL = N - CHUNK * NW
TBL = 32


def _gather_body(z_hbm, scale_hbm, shift_hbm, s_hbm, t_hbm,
                 z_v, s_v, t_v, scale_v, shift_v, tz_v, ts_v, tt_v):
    wid = lax.axis_index("s") * NC + lax.axis_index("c")
    base = wid * CHUNK

    pltpu.sync_copy(scale_hbm, scale_v)
    pltpu.sync_copy(shift_hbm, shift_v)

    pltpu.sync_copy(z_hbm.at[pl.ds(base, CHUNK)], z_v)

    def body(i, _):
        off = i * L
        zv = z_v[pl.ds(off, L)]
        s_v[pl.ds(off, L)] = plsc.load_gather(scale_v, [zv])
        t_v[pl.ds(off, L)] = plsc.load_gather(shift_v, [zv])
        return _

    lax.fori_loop(0, CHUNK // L, body, None, unroll=8)

    pltpu.sync_copy(s_v, s_hbm.at[pl.ds(base, CHUNK)])
    pltpu.sync_copy(t_v, t_hbm.at[pl.ds(base, CHUNK)])

    @pl.when(wid == NW - 1)
    def _():
        tbase = CHUNK * NW
        pltpu.sync_copy(z_hbm.at[pl.ds(tbase, TAIL)], tz_v)

        def tbody(i, _):
            off = i * L
            zv = tz_v[pl.ds(off, L)]
            ts_v[pl.ds(off, L)] = plsc.load_gather(scale_v, [zv])
            tt_v[pl.ds(off, L)] = plsc.load_gather(shift_v, [zv])
            return _

        lax.fori_loop(0, TAIL // L, tbody, None, unroll=TAIL // L)
        pltpu.sync_copy(ts_v, s_hbm.at[pl.ds(tbase, TAIL)])
        pltpu.sync_copy(tt_v, t_hbm.at[pl.ds(tbase, TAIL)])


@jax.jit
def kernel(inputs, z, scale_table, shift_table):
    zi = z.astype(jnp.int32)
    zmax = scale_table.shape[0]
    scale = jnp.zeros((TBL,), jnp.float32).at[:zmax].set(scale_table.reshape(-1))
    shift = jnp.zeros((TBL,), jnp.float32).at[:zmax].set(shift_table.reshape(-1))

    mesh = plsc.VectorSubcoreMesh(core_axis_name="c", subcore_axis_name="s",
                                  num_cores=NC, num_subcores=NS)
    s, t = pl.kernel(
        _gather_body,
        out_type=(jax.ShapeDtypeStruct((N,), jnp.float32),
                  jax.ShapeDtypeStruct((N,), jnp.float32)),
        mesh=mesh,
        compiler_params=pltpu.CompilerParams(needs_layout_passes=False),
        scratch_types=[
            pltpu.VMEM((CHUNK,), jnp.int32),
            pltpu.VMEM((CHUNK,), jnp.float32),
            pltpu.VMEM((CHUNK,), jnp.float32),
            pltpu.VMEM((TBL,), jnp.float32),
            pltpu.VMEM((TBL,), jnp.float32),
            pltpu.VMEM((TAIL,), jnp.int32),
            pltpu.VMEM((TAIL,), jnp.float32),
            pltpu.VMEM((TAIL,), jnp.float32),
        ],
    )(zi, scale, shift)
    return inputs * s[:, None] + t[:, None]


# speedup vs baseline: 2.0945x; 1.3517x over previous
import jax
import jax.numpy as jnp
from jax import lax
from jax.experimental import pallas as pl
from jax.experimental.pallas import tpu as pltpu
from jax.experimental.pallas import tpu_sc as plsc

N = 1_000_000
NC = 2
NS = 16
NW = NC * NS
L = 16
G = 8

CHUNK = (N // NW) // (G * L) * (G * L)
TAIL = N - CHUNK * NW
TBL = 18


def _gather_group(z_ref, s_ref, t_ref, scale_v, shift_v, off, n_chains):
    zs = [z_ref[pl.ds(off + k * L, L)] for k in range(n_chains)]
    svs = [plsc.load_gather(scale_v, [zv]) for zv in zs]
    tvs = [plsc.load_gather(shift_v, [zv]) for zv in zs]
    for k in range(n_chains):
        s_ref[pl.ds(off + k * L, L)] = svs[k]
        t_ref[pl.ds(off + k * L, L)] = tvs[k]


def _gather_body(z_hbm, scale_hbm, shift_hbm, s_hbm, t_hbm,
                 z_v, s_v, t_v, scale_v, shift_v):
    wid = lax.axis_index("s") * NC + lax.axis_index("c")
    base = wid * CHUNK

    pltpu.sync_copy(scale_hbm, scale_v)
    pltpu.sync_copy(shift_hbm, shift_v)

    pltpu.sync_copy(z_hbm.at[pl.ds(base, CHUNK)], z_v)

    def body(i, _):
        _gather_group(z_v, s_v, t_v, scale_v, shift_v, i * (G * L), G)
        return _

    lax.fori_loop(0, CHUNK // (G * L), body, None)

    pltpu.sync_copy(s_v, s_hbm.at[pl.ds(base, CHUNK)])
    pltpu.sync_copy(t_v, t_hbm.at[pl.ds(base, CHUNK)])

    @pl.when(wid == NW - 1)
    def _():
        tbase = CHUNK * NW
        pltpu.sync_copy(z_hbm.at[pl.ds(tbase, TAIL)], z_v.at[pl.ds(0, TAIL)])

        def tbody(i, _):
            _gather_group(z_v, s_v, t_v, scale_v, shift_v, i * (G * L), G)
            return _

        lax.fori_loop(0, TAIL // (G * L), tbody, None)
        for j in range(TAIL // (G * L) * G, TAIL // L):
            _gather_group(z_v, s_v, t_v, scale_v, shift_v, j * L, 1)
        pltpu.sync_copy(s_v.at[pl.ds(0, TAIL)], s_hbm.at[pl.ds(tbase, TAIL)])
        pltpu.sync_copy(t_v.at[pl.ds(0, TAIL)], t_hbm.at[pl.ds(tbase, TAIL)])


@jax.jit
def kernel(inputs, z, scale_table, shift_table):
    zi = z.astype(jnp.int32)
    scale = scale_table.reshape(TBL)
    shift = shift_table.reshape(TBL)

    mesh = plsc.VectorSubcoreMesh(core_axis_name="c", subcore_axis_name="s",
                                  num_cores=NC, num_subcores=NS)
    s, t = pl.kernel(
        _gather_body,
        out_type=(jax.ShapeDtypeStruct((N,), jnp.float32),
                  jax.ShapeDtypeStruct((N,), jnp.float32)),
        mesh=mesh,
        compiler_params=pltpu.CompilerParams(needs_layout_passes=False),
        scratch_types=[
            pltpu.VMEM((CHUNK,), jnp.int32),
            pltpu.VMEM((CHUNK,), jnp.float32),
            pltpu.VMEM((CHUNK,), jnp.float32),
            pltpu.VMEM((TBL,), jnp.float32),
            pltpu.VMEM((TBL,), jnp.float32),
        ],
    )(zi, scale, shift)
    return inputs * s[:, None] + t[:, None]
